# baseline (speedup 1.0000x reference)
import jax
import jax.numpy as jnp
from jax.experimental import pallas as pl


_ROWS = 32 * 1 * 1024
_COLS = 256
_BLOCK_ROWS = 4096
_GRID = _ROWS // _BLOCK_ROWS


def _loss_body(hard_ref, soft_ref, logsum_ref, count_ref):
    i = pl.program_id(0)

    @pl.when(i == 0)
    def _init():
        logsum_ref[...] = jnp.zeros_like(logsum_ref)
        count_ref[...] = jnp.zeros_like(count_ref)

    h = hard_ref[...]
    s = soft_ref[...]
    logsum_ref[...] += jnp.sum(h * jnp.log(s)).reshape(1, 1)
    count_ref[...] += jnp.sum(h).reshape(1, 1)


def kernel(hard_attention, soft_attention):
    h2 = hard_attention.reshape(_ROWS, _COLS)
    s2 = soft_attention.reshape(_ROWS, _COLS)
    logsum, count = pl.pallas_call(
        _loss_body,
        grid=(_GRID,),
        in_specs=[
            pl.BlockSpec((_BLOCK_ROWS, _COLS), lambda i: (i, 0)),
            pl.BlockSpec((_BLOCK_ROWS, _COLS), lambda i: (i, 0)),
        ],
        out_specs=[
            pl.BlockSpec((1, 1), lambda i: (0, 0)),
            pl.BlockSpec((1, 1), lambda i: (0, 0)),
        ],
        out_shape=[
            jax.ShapeDtypeStruct((1, 1), jnp.float32),
            jax.ShapeDtypeStruct((1, 1), jnp.float32),
        ],
    )(h2, s2)
    return -logsum[0, 0] / count[0, 0]
